# baseline (speedup 1.0000x reference)
import functools

import jax
import jax.numpy as jnp
from jax import lax
from jax.experimental import pallas as pl
from jax.experimental.pallas import tpu as pltpu
from jax.experimental.pallas import tpu_sc as plsc

NC = 2
NS = 16
CH = 128
SPLIT0 = 65.0 / 160.0


def _deg_call(Npad, na, nb):
    mesh = plsc.VectorSubcoreMesh(
        core_axis_name="c", subcore_axis_name="s",
        num_cores=NC, num_subcores=NS)
    rpt = Npad // NS

    @functools.partial(
        pl.kernel,
        out_type=jax.ShapeDtypeStruct((NC, Npad), jnp.float32),
        mesh=mesh,
        scratch_types=[
            pltpu.VMEM((max(na, nb), CH), jnp.int32),
            pltpu.VMEM((CH,), jnp.float32),
            pltpu.VMEM_SHARED((Npad,), jnp.float32),
        ],
    )
    def deg_kernel(dst0_hbm, dst1_hbm, ones_hbm, zeros_hbm, out_hbm,
                   idx_v, ones_v, acc):
        c = lax.axis_index("c")
        s = lax.axis_index("s")
        r0 = s * rpt
        pltpu.sync_copy(zeros_hbm.at[pl.ds(r0, rpt)], acc.at[pl.ds(r0, rpt)])
        pltpu.sync_copy(ones_hbm, ones_v)

        @pl.when(c == 0)
        def _():
            pltpu.sync_copy(dst0_hbm.at[s], idx_v.at[pl.ds(0, na)])

        @pl.when(c == 1)
        def _():
            pltpu.sync_copy(dst1_hbm.at[s], idx_v.at[pl.ds(0, nb)])

        plsc.subcore_barrier()
        nloc = jnp.where(c == 0, na, nb)

        @pl.loop(0, nloc)
        def _(j):
            pltpu.sync_copy(ones_v, acc.at[idx_v.at[j]], add=True)

        plsc.subcore_barrier()
        pltpu.sync_copy(acc.at[pl.ds(r0, rpt)], out_hbm.at[c, pl.ds(r0, rpt)])

    return deg_kernel


def _agg_call(N, D, Npad, na, nb):
    mesh = plsc.VectorSubcoreMesh(
        core_axis_name="c", subcore_axis_name="s",
        num_cores=NC, num_subcores=NS)
    rpt = Npad // NS

    @functools.partial(
        pl.kernel,
        out_type=jax.ShapeDtypeStruct((NC, Npad, D), jnp.float32),
        mesh=mesh,
        scratch_types=[
            pltpu.VMEM((max(na, nb), CH), jnp.int32),
            pltpu.VMEM((max(na, nb), CH), jnp.int32),
            pltpu.VMEM((CH, D), jnp.float32),
            pltpu.VMEM_SHARED((Npad, D), jnp.float32),
            pltpu.SemaphoreType.DMA,
            pltpu.SemaphoreType.DMA,
        ],
    )
    def agg_kernel(y_hbm, src0_hbm, dst0_hbm, src1_hbm, dst1_hbm,
                   out_hbm, src_v, dst_v, rows_v, acc, gsem, isem):
        c = lax.axis_index("c")
        s = lax.axis_index("s")
        r0 = s * rpt

        @pl.loop(0, CH)
        def _(i):
            for k in range(D // 16):
                rows_v[i, pl.ds(k * 16, 16)] = jnp.zeros((16,), jnp.float32)

        @pl.when(c == 0)
        def _():
            pltpu.async_copy(src0_hbm.at[s], src_v.at[pl.ds(0, na)], isem)
            pltpu.async_copy(dst0_hbm.at[s], dst_v.at[pl.ds(0, na)], isem)

        @pl.when(c == 1)
        def _():
            pltpu.async_copy(src1_hbm.at[s], src_v.at[pl.ds(0, nb)], isem)
            pltpu.async_copy(dst1_hbm.at[s], dst_v.at[pl.ds(0, nb)], isem)

        @pl.loop(0, rpt // CH)
        def _(r):
            pltpu.sync_copy(rows_v, acc.at[pl.ds(r0 + r * CH, CH)])

        @pl.when(c == 0)
        def _():
            pltpu.make_async_copy(src0_hbm.at[s], src_v.at[pl.ds(0, na)],
                                  isem).wait()
            pltpu.make_async_copy(dst0_hbm.at[s], dst_v.at[pl.ds(0, na)],
                                  isem).wait()

        @pl.when(c == 1)
        def _():
            pltpu.make_async_copy(src1_hbm.at[s], src_v.at[pl.ds(0, nb)],
                                  isem).wait()
            pltpu.make_async_copy(dst1_hbm.at[s], dst_v.at[pl.ds(0, nb)],
                                  isem).wait()

        plsc.subcore_barrier()
        nloc = jnp.where(c == 0, na, nb)

        @pl.loop(0, nloc)
        def _(j):
            h = CH // 4
            ds_ = [pltpu.async_copy(y_hbm.at[src_v.at[j, pl.ds(k * h, h)]],
                                    rows_v.at[pl.ds(k * h, h)], gsem)
                   for k in range(4)]
            for d in ds_:
                d.wait()
            pltpu.sync_copy(rows_v, acc.at[dst_v.at[j]], add=True)

        plsc.subcore_barrier()
        pltpu.sync_copy(acc.at[pl.ds(r0, rpt)], out_hbm.at[c, pl.ds(r0, rpt)])

    return agg_kernel


def _xw_call(N, D, R):
    def body(x_ref, w_ref, y_ref):
        y_ref[...] = jnp.dot(x_ref[...], w_ref[...],
                             preferred_element_type=jnp.float32)

    return pl.pallas_call(
        body,
        grid=(N // R,),
        in_specs=[
            pl.BlockSpec((R, D), lambda i: (i, 0)),
            pl.BlockSpec((D, D), lambda i: (0, 0)),
        ],
        out_specs=pl.BlockSpec((R, D), lambda i: (i, 0)),
        out_shape=jax.ShapeDtypeStruct((N, D), jnp.float32),
    )


def _scale_call(N, D, R):
    def body(xw_ref, deg_ref, y_ref):
        deg = deg_ref[:, 0:1] + deg_ref[:, 1:2] + 1.0
        y_ref[...] = xw_ref[...] * lax.rsqrt(deg)

    return pl.pallas_call(
        body,
        grid=(N // R,),
        in_specs=[
            pl.BlockSpec((R, D), lambda i: (i, 0)),
            pl.BlockSpec((R, NC), lambda i: (i, 0)),
        ],
        out_specs=pl.BlockSpec((R, D), lambda i: (i, 0)),
        out_shape=jax.ShapeDtypeStruct((N, D), jnp.float32),
    )


def _epilogue_call(N, D, R):
    def body(agg_ref, y_ref, x_ref, deg_ref, b_ref, g_ref, bt_ref, a_ref,
             out_ref):
        agg0 = agg_ref[0] + agg_ref[1] + y_ref[...]
        deg = deg_ref[:, 0:1] + deg_ref[:, 1:2] + 1.0
        h = agg0 * lax.rsqrt(deg) + b_ref[...]
        mu = jnp.mean(h, axis=-1, keepdims=True)
        d = h - mu
        var = jnp.mean(d * d, axis=-1, keepdims=True)
        ln = d * lax.rsqrt(var + 1e-5) * g_ref[...] + bt_ref[...]
        out_ref[...] = jnp.where(ln >= 0, ln, a_ref[0, 0] * ln) + x_ref[...]

    return pl.pallas_call(
        body,
        grid=(N // R,),
        in_specs=[
            pl.BlockSpec((NC, R, D), lambda i: (0, i, 0)),
            pl.BlockSpec((R, D), lambda i: (i, 0)),
            pl.BlockSpec((R, D), lambda i: (i, 0)),
            pl.BlockSpec((R, NC), lambda i: (i, 0)),
            pl.BlockSpec((1, D), lambda i: (0, 0)),
            pl.BlockSpec((1, D), lambda i: (0, 0)),
            pl.BlockSpec((1, D), lambda i: (0, 0)),
            pl.BlockSpec((1, 1), lambda i: (0, 0)),
        ],
        out_specs=pl.BlockSpec((R, D), lambda i: (i, 0)),
        out_shape=jax.ShapeDtypeStruct((N, D), jnp.float32),
    )


def kernel(x, edge_index, W, b, ln_gamma, ln_beta, prelu_a):
    N, D = x.shape
    E = edge_index.shape[1]

    Npad = ((N + 1 + 255) // 256) * 256
    tot = -(-E // (NS * CH))
    na = max(1, min(tot - 1, int(round(tot * SPLIT0))))
    nb = tot - na
    Epad = NS * tot * CH

    src = edge_index[0]
    dst = edge_index[1]
    pad = Epad - E
    src_p = jnp.concatenate([src, jnp.zeros((pad,), jnp.int32)])
    dst_p = jnp.concatenate([dst, jnp.full((pad,), N, jnp.int32)])
    cut = NS * na * CH
    src0 = src_p[:cut].reshape(NS, na, CH)
    dst0 = dst_p[:cut].reshape(NS, na, CH)
    src1 = src_p[cut:].reshape(NS, nb, CH)
    dst1 = dst_p[cut:].reshape(NS, nb, CH)

    ones1 = jnp.ones((CH,), jnp.float32)
    zeros1 = jnp.zeros((Npad,), jnp.float32)

    R = 1000
    xw = _xw_call(N, D, R)(x, W)
    deg_p = _deg_call(Npad, na, nb)(dst0, dst1, ones1, zeros1)
    deg_t = deg_p.T
    y = _scale_call(N, D, R)(xw, deg_t)
    agg_p = _agg_call(N, D, Npad, na, nb)(y, src0, dst0, src1, dst1)
    out = _epilogue_call(N, D, R)(
        agg_p, y, x, deg_t,
        b.reshape(1, D), ln_gamma.reshape(1, D), ln_beta.reshape(1, D),
        prelu_a.reshape(1, 1))
    return out


# speedup vs baseline: 1.0157x; 1.0157x over previous
import functools

import jax
import jax.numpy as jnp
from jax import lax
from jax.experimental import pallas as pl
from jax.experimental.pallas import tpu as pltpu
from jax.experimental.pallas import tpu_sc as plsc

NC = 2
NS = 16
CH = 128
SPLIT0 = 65.0 / 160.0


def _deg_call(Npad, na, nb):
    mesh = plsc.VectorSubcoreMesh(
        core_axis_name="c", subcore_axis_name="s",
        num_cores=NC, num_subcores=NS)
    rpt = Npad // NS

    @functools.partial(
        pl.kernel,
        out_type=jax.ShapeDtypeStruct((NC, Npad), jnp.float32),
        mesh=mesh,
        scratch_types=[
            pltpu.VMEM((max(na, nb), CH), jnp.int32),
            pltpu.VMEM((CH,), jnp.float32),
            pltpu.VMEM_SHARED((Npad,), jnp.float32),
        ],
    )
    def deg_kernel(dst0_hbm, dst1_hbm, ones_hbm, zeros_hbm, out_hbm,
                   idx_v, ones_v, acc):
        c = lax.axis_index("c")
        s = lax.axis_index("s")
        r0 = s * rpt
        pltpu.sync_copy(zeros_hbm.at[pl.ds(r0, rpt)], acc.at[pl.ds(r0, rpt)])
        pltpu.sync_copy(ones_hbm, ones_v)

        @pl.when(c == 0)
        def _():
            pltpu.sync_copy(dst0_hbm.at[s], idx_v.at[pl.ds(0, na)])

        @pl.when(c == 1)
        def _():
            pltpu.sync_copy(dst1_hbm.at[s], idx_v.at[pl.ds(0, nb)])

        plsc.subcore_barrier()
        nloc = jnp.where(c == 0, na, nb)

        @pl.loop(0, nloc)
        def _(j):
            pltpu.sync_copy(ones_v, acc.at[idx_v.at[j]], add=True)

        plsc.subcore_barrier()
        pltpu.sync_copy(acc.at[pl.ds(r0, rpt)], out_hbm.at[c, pl.ds(r0, rpt)])

    return deg_kernel


def _agg_call(N, D, Npad, na, nb):
    mesh = plsc.VectorSubcoreMesh(
        core_axis_name="c", subcore_axis_name="s",
        num_cores=NC, num_subcores=NS)
    rpt = Npad // NS

    @functools.partial(
        pl.kernel,
        out_type=jax.ShapeDtypeStruct((NC, Npad, D), jnp.float32),
        mesh=mesh,
        scratch_types=[
            pltpu.VMEM((max(na, nb), CH), jnp.int32),
            pltpu.VMEM((max(na, nb), CH), jnp.int32),
            pltpu.VMEM((CH, D), jnp.float32),
            pltpu.VMEM_SHARED((Npad, D), jnp.float32),
            pltpu.SemaphoreType.DMA,
            pltpu.SemaphoreType.DMA,
        ],
    )
    def agg_kernel(y_hbm, src0_hbm, dst0_hbm, src1_hbm, dst1_hbm,
                   out_hbm, src_v, dst_v, rows_v, acc, gsem, isem):
        c = lax.axis_index("c")
        s = lax.axis_index("s")
        r0 = s * rpt

        @pl.loop(0, CH)
        def _(i):
            for k in range(D // 16):
                rows_v[i, pl.ds(k * 16, 16)] = jnp.zeros((16,), jnp.float32)

        @pl.when(c == 0)
        def _():
            pltpu.async_copy(src0_hbm.at[s], src_v.at[pl.ds(0, na)], isem)
            pltpu.async_copy(dst0_hbm.at[s], dst_v.at[pl.ds(0, na)], isem)

        @pl.when(c == 1)
        def _():
            pltpu.async_copy(src1_hbm.at[s], src_v.at[pl.ds(0, nb)], isem)
            pltpu.async_copy(dst1_hbm.at[s], dst_v.at[pl.ds(0, nb)], isem)

        @pl.loop(0, rpt // CH)
        def _(r):
            pltpu.sync_copy(rows_v, acc.at[pl.ds(r0 + r * CH, CH)])

        @pl.when(c == 0)
        def _():
            pltpu.make_async_copy(src0_hbm.at[s], src_v.at[pl.ds(0, na)],
                                  isem).wait()
            pltpu.make_async_copy(dst0_hbm.at[s], dst_v.at[pl.ds(0, na)],
                                  isem).wait()

        @pl.when(c == 1)
        def _():
            pltpu.make_async_copy(src1_hbm.at[s], src_v.at[pl.ds(0, nb)],
                                  isem).wait()
            pltpu.make_async_copy(dst1_hbm.at[s], dst_v.at[pl.ds(0, nb)],
                                  isem).wait()

        plsc.subcore_barrier()
        nloc = jnp.where(c == 0, na, nb)

        @pl.loop(0, nloc)
        def _(j):
            h = CH // 2
            d1 = pltpu.async_copy(y_hbm.at[src_v.at[j, pl.ds(0, h)]],
                                  rows_v.at[pl.ds(0, h)], gsem)
            d2 = pltpu.async_copy(y_hbm.at[src_v.at[j, pl.ds(h, h)]],
                                  rows_v.at[pl.ds(h, h)], isem)
            d1.wait()
            d2.wait()
            pltpu.sync_copy(rows_v, acc.at[dst_v.at[j]], add=True)

        plsc.subcore_barrier()
        pltpu.sync_copy(acc.at[pl.ds(r0, rpt)], out_hbm.at[c, pl.ds(r0, rpt)])

    return agg_kernel


def _xw_call(N, D, R):
    def body(x_ref, w_ref, y_ref):
        y_ref[...] = jnp.dot(x_ref[...], w_ref[...],
                             preferred_element_type=jnp.float32)

    return pl.pallas_call(
        body,
        grid=(N // R,),
        in_specs=[
            pl.BlockSpec((R, D), lambda i: (i, 0)),
            pl.BlockSpec((D, D), lambda i: (0, 0)),
        ],
        out_specs=pl.BlockSpec((R, D), lambda i: (i, 0)),
        out_shape=jax.ShapeDtypeStruct((N, D), jnp.float32),
    )


def _scale_call(N, D, R):
    def body(xw_ref, deg_ref, y_ref):
        deg = deg_ref[:, 0:1] + deg_ref[:, 1:2] + 1.0
        y_ref[...] = xw_ref[...] * lax.rsqrt(deg)

    return pl.pallas_call(
        body,
        grid=(N // R,),
        in_specs=[
            pl.BlockSpec((R, D), lambda i: (i, 0)),
            pl.BlockSpec((R, NC), lambda i: (i, 0)),
        ],
        out_specs=pl.BlockSpec((R, D), lambda i: (i, 0)),
        out_shape=jax.ShapeDtypeStruct((N, D), jnp.float32),
    )


def _epilogue_call(N, D, R):
    def body(agg_ref, y_ref, x_ref, deg_ref, b_ref, g_ref, bt_ref, a_ref,
             out_ref):
        agg0 = agg_ref[0] + agg_ref[1] + y_ref[...]
        deg = deg_ref[:, 0:1] + deg_ref[:, 1:2] + 1.0
        h = agg0 * lax.rsqrt(deg) + b_ref[...]
        mu = jnp.mean(h, axis=-1, keepdims=True)
        d = h - mu
        var = jnp.mean(d * d, axis=-1, keepdims=True)
        ln = d * lax.rsqrt(var + 1e-5) * g_ref[...] + bt_ref[...]
        out_ref[...] = jnp.where(ln >= 0, ln, a_ref[0, 0] * ln) + x_ref[...]

    return pl.pallas_call(
        body,
        grid=(N // R,),
        in_specs=[
            pl.BlockSpec((NC, R, D), lambda i: (0, i, 0)),
            pl.BlockSpec((R, D), lambda i: (i, 0)),
            pl.BlockSpec((R, D), lambda i: (i, 0)),
            pl.BlockSpec((R, NC), lambda i: (i, 0)),
            pl.BlockSpec((1, D), lambda i: (0, 0)),
            pl.BlockSpec((1, D), lambda i: (0, 0)),
            pl.BlockSpec((1, D), lambda i: (0, 0)),
            pl.BlockSpec((1, 1), lambda i: (0, 0)),
        ],
        out_specs=pl.BlockSpec((R, D), lambda i: (i, 0)),
        out_shape=jax.ShapeDtypeStruct((N, D), jnp.float32),
    )


def kernel(x, edge_index, W, b, ln_gamma, ln_beta, prelu_a):
    N, D = x.shape
    E = edge_index.shape[1]

    Npad = ((N + 1 + 255) // 256) * 256
    tot = -(-E // (NS * CH))
    na = max(1, min(tot - 1, int(round(tot * SPLIT0))))
    nb = tot - na
    Epad = NS * tot * CH

    src = edge_index[0]
    dst = edge_index[1]
    pad = Epad - E
    src_p = jnp.concatenate([src, jnp.zeros((pad,), jnp.int32)])
    dst_p = jnp.concatenate([dst, jnp.full((pad,), N, jnp.int32)])
    cut = NS * na * CH
    src0 = src_p[:cut].reshape(NS, na, CH)
    dst0 = dst_p[:cut].reshape(NS, na, CH)
    src1 = src_p[cut:].reshape(NS, nb, CH)
    dst1 = dst_p[cut:].reshape(NS, nb, CH)

    ones1 = jnp.ones((CH,), jnp.float32)
    zeros1 = jnp.zeros((Npad,), jnp.float32)

    R = 1000
    xw = _xw_call(N, D, R)(x, W)
    deg_p = _deg_call(Npad, na, nb)(dst0, dst1, ones1, zeros1)
    deg_t = deg_p.T
    y = _scale_call(N, D, R)(xw, deg_t)
    agg_p = _agg_call(N, D, Npad, na, nb)(y, src0, dst0, src1, dst1)
    out = _epilogue_call(N, D, R)(
        agg_p, y, x, deg_t,
        b.reshape(1, D), ln_gamma.reshape(1, D), ln_beta.reshape(1, D),
        prelu_a.reshape(1, 1))
    return out
